# baseline (speedup 1.0000x reference)
import functools

import jax
import jax.numpy as jnp
from jax import lax
from jax.experimental import pallas as pl
from jax.experimental.pallas import tpu as pltpu
from jax.experimental.pallas import tpu_sc as plsc

_N = 10000
_NP = 10240
_D = 128
_T = 5
_NG = 128
_E = 320000
_EP = 327680
_NCORES = 2
_NSUB = 16
_NW = _NCORES * _NSUB
_TRASH = _NP - 8

_BLK = 1024
_NBLK = _NP // _BLK

_ECH = 128
_NCH = 64
_SR = 160


def _sc_mesh():
    return plsc.VectorSubcoreMesh(
        core_axis_name="c", subcore_axis_name="s",
        num_cores=_NCORES, num_subcores=_NSUB)


def _embed_body(tidx, aidx, didx, temb, aemb, demb, h0_out,
                idx_v, ra, rb, rc, sem):
    c = lax.axis_index("c")
    s = lax.axis_index("s")
    w = s * _NCORES + c
    per_w = _NP // _NW
    base = w * per_w

    def chunk(i, carry):
        off = base + i * _NCH
        pltpu.sync_copy(tidx.at[pl.ds(off, _NCH)], idx_v)
        pltpu.async_copy(temb.at[idx_v], ra, sem).wait()
        pltpu.sync_copy(aidx.at[pl.ds(off, _NCH)], idx_v)
        pltpu.async_copy(aemb.at[idx_v], rb, sem).wait()
        pltpu.sync_copy(didx.at[pl.ds(off, _NCH)], idx_v)
        pltpu.async_copy(demb.at[idx_v], rc, sem).wait()

        def add_b(k, carry2):
            r = k // (_D // 16)
            col = (k % (_D // 16)) * 16
            ra[r, pl.ds(col, 16)] = (ra[r, pl.ds(col, 16)]
                                     + rb[r, pl.ds(col, 16)]
                                     + rc[r, pl.ds(col, 16)])
            return carry2

        lax.fori_loop(0, _NCH * (_D // 16), add_b, 0)
        pltpu.sync_copy(ra, h0_out.at[pl.ds(off, _NCH)])
        return carry

    lax.fori_loop(0, per_w // _NCH, chunk, 0)


def _embed_call(tidx, aidx, didx, temb, aemb, demb):
    f = pl.kernel(
        _embed_body,
        out_type=jax.ShapeDtypeStruct((_NP, _D), jnp.float32),
        mesh=_sc_mesh(),
        scratch_types=[
            pltpu.VMEM((_NCH,), jnp.int32),
            pltpu.VMEM((_NCH, _D), jnp.float32),
            pltpu.VMEM((_NCH, _D), jnp.float32),
            pltpu.VMEM((_NCH, _D), jnp.float32),
            pltpu.SemaphoreType.DMA,
        ],
    )
    return f(tidx, aidx, didx, temb, aemb, demb)


def _scatter_body(src, dst, m, zrows, p0, p1,
                  acc, sidx0, sidx1, didx0, didx1, rows0, rows1, sem0, sem1):
    c = lax.axis_index("c")
    s = lax.axis_index("s")
    w = s * _NCORES + c
    zper = _NP // _NSUB
    zbase = s * zper
    nchunk = (_EP // _NW) // _ECH
    cbase = w * nchunk

    pltpu.sync_copy(zrows, rows0)

    def zb(j, carry):
        pltpu.sync_copy(rows0, acc.at[pl.ds(zbase + j * _ECH, _ECH)])
        return carry

    lax.fori_loop(0, zper // _ECH, zb, 0)
    plsc.subcore_barrier()

    pltpu.sync_copy(src.at[cbase], sidx0)
    pltpu.sync_copy(dst.at[cbase], didx0)
    pltpu.async_copy(m.at[sidx0], rows0, sem0)

    def chunk2(i, carry):
        j = i * 2
        pltpu.sync_copy(src.at[cbase + j + 1], sidx1)
        pltpu.sync_copy(dst.at[cbase + j + 1], didx1)
        pltpu.async_copy(m.at[sidx1], rows1, sem1)
        pltpu.make_async_copy(m.at[sidx0], rows0, sem0).wait()
        pltpu.sync_copy(rows0, acc.at[didx0], add=True)

        @pl.when(j + 2 < nchunk)
        def _():
            pltpu.sync_copy(src.at[cbase + j + 2], sidx0)
            pltpu.sync_copy(dst.at[cbase + j + 2], didx0)
            pltpu.async_copy(m.at[sidx0], rows0, sem0)

        pltpu.make_async_copy(m.at[sidx1], rows1, sem1).wait()
        pltpu.sync_copy(rows1, acc.at[didx1], add=True)
        return carry

    lax.fori_loop(0, nchunk // 2, chunk2, 0)
    plsc.subcore_barrier()

    def wb(j, carry):
        r0 = zbase + j * _ECH
        pltpu.sync_copy(acc.at[pl.ds(r0, _ECH)], rows0)

        @pl.when(c == 0)
        def _():
            pltpu.sync_copy(rows0, p0.at[pl.ds(r0, _ECH)])

        @pl.when(c == 1)
        def _():
            pltpu.sync_copy(rows0, p1.at[pl.ds(r0, _ECH)])

        return carry

    lax.fori_loop(0, zper // _ECH, wb, 0)


def _scatter_call(src, dst, m, zrows):
    f = pl.kernel(
        _scatter_body,
        out_type=(jax.ShapeDtypeStruct((_NP, _D), jnp.float32),
                  jax.ShapeDtypeStruct((_NP, _D), jnp.float32)),
        mesh=_sc_mesh(),
        scratch_types=[
            pltpu.VMEM_SHARED((_NP, _D), jnp.float32),
            pltpu.VMEM((_ECH,), jnp.int32),
            pltpu.VMEM((_ECH,), jnp.int32),
            pltpu.VMEM((_ECH,), jnp.int32),
            pltpu.VMEM((_ECH,), jnp.int32),
            pltpu.VMEM((_ECH, _D), jnp.float32),
            pltpu.VMEM((_ECH, _D), jnp.float32),
            pltpu.SemaphoreType.DMA,
            pltpu.SemaphoreType.DMA,
        ],
    )
    return f(src, dst, m, zrows)


def _mm_body(a, b, o):
    o[...] = jnp.dot(a[...], b[...], preferred_element_type=jnp.float32)


def _mm_call(a, b):
    return pl.pallas_call(
        _mm_body,
        grid=(_NBLK,),
        in_specs=[
            pl.BlockSpec((_BLK, _D), lambda i: (i, 0)),
            pl.BlockSpec((_D, _D), lambda i: (0, 0)),
        ],
        out_specs=pl.BlockSpec((_BLK, _D), lambda i: (i, 0)),
        out_shape=jax.ShapeDtypeStruct((_NP, _D), jnp.float32),
    )(a, b)


def _gru_body(p0, p1, h, wih, whh, bih, bhh, wn, h_out, m_out):
    agg = p0[...] + p1[...]
    gi = jnp.dot(agg, wih[...], preferred_element_type=jnp.float32) + bih[...]
    gh = jnp.dot(h[...], whh[...], preferred_element_type=jnp.float32) + bhh[...]
    r = jax.nn.sigmoid(gi[:, :_D] + gh[:, :_D])
    z = jax.nn.sigmoid(gi[:, _D:2 * _D] + gh[:, _D:2 * _D])
    n = jnp.tanh(gi[:, 2 * _D:] + r * gh[:, 2 * _D:])
    hn = (1.0 - z) * n + z * h[...]
    h_out[...] = hn
    m_out[...] = jnp.dot(hn, wn[...], preferred_element_type=jnp.float32)


def _gru_call(p0, p1, h, wih, whh, bih, bhh, wn):
    row = pl.BlockSpec((_BLK, _D), lambda i: (i, 0))
    return pl.pallas_call(
        _gru_body,
        grid=(_NBLK,),
        in_specs=[
            row, row, row,
            pl.BlockSpec((_D, 3 * _D), lambda i: (0, 0)),
            pl.BlockSpec((_D, 3 * _D), lambda i: (0, 0)),
            pl.BlockSpec((1, 3 * _D), lambda i: (0, 0)),
            pl.BlockSpec((1, 3 * _D), lambda i: (0, 0)),
            pl.BlockSpec((_D, _D), lambda i: (0, 0)),
        ],
        out_specs=(row, row),
        out_shape=(jax.ShapeDtypeStruct((_NP, _D), jnp.float32),
                   jax.ShapeDtypeStruct((_NP, _D), jnp.float32)),
    )(p0, p1, h, wih, whh, bih, bhh, wn)


def _final_body(h, h0, bt, clwh, clwh0, clb, crwh, crwh0, crb, pw, pb,
                pooled, out):
    pid = pl.program_id(0)

    @pl.when(pid == 0)
    def _():
        pooled[...] = jnp.zeros_like(pooled)

    a = (jnp.dot(h[...], clwh[...], preferred_element_type=jnp.float32)
         + jnp.dot(h0[...], clwh0[...], preferred_element_type=jnp.float32)
         + clb[...])
    b = (jnp.dot(h[...], crwh[...], preferred_element_type=jnp.float32)
         + jnp.dot(h0[...], crwh0[...], preferred_element_type=jnp.float32)
         + crb[...])
    g = jax.nn.sigmoid(a) * jnp.tanh(b)
    lbl = bt[0, 0, :]
    mask = (lbl[None, :] == lax.broadcasted_iota(
        jnp.int32, (_NG, _BLK), 0)).astype(jnp.float32)
    pooled[...] += jnp.dot(mask, g, preferred_element_type=jnp.float32)

    @pl.when(pid == pl.num_programs(0) - 1)
    def _():
        out[...] = (jnp.dot(pooled[...], pw[...],
                            preferred_element_type=jnp.float32) + pb[...])


def _final_call(h, h0, bt, clwh, clwh0, clb, crwh, crwh0, crb, pw, pb):
    row = pl.BlockSpec((_BLK, _D), lambda i: (i, 0))
    full = pl.BlockSpec((_D, _D), lambda i: (0, 0))
    bias = pl.BlockSpec((1, _D), lambda i: (0, 0))
    pooled, out = pl.pallas_call(
        _final_body,
        grid=(_NBLK,),
        in_specs=[
            row, row,
            pl.BlockSpec((1, 1, _BLK), lambda i: (i, 0, 0)),
            full, full, bias, full, full, bias, full, bias,
        ],
        out_specs=(pl.BlockSpec((_NG, _D), lambda i: (0, 0)),
                   pl.BlockSpec((_NG, _D), lambda i: (0, 0))),
        out_shape=(jax.ShapeDtypeStruct((_NG, _D), jnp.float32),
                   jax.ShapeDtypeStruct((_NG, _D), jnp.float32)),
    )(h, h0, bt, clwh, clwh0, clb, crwh, crwh0, crb, pw, pb)
    return out


def kernel(x, node_depth, edge_index, batch, type_emb, attr_emb, depth_emb,
           ggnn_w, w_ih, w_hh, b_ih, b_hh, cl_w, cl_b, cr_w, cr_b,
           pred_w, pred_b):
    f32 = jnp.float32
    padn = _NP - _N
    tidx = jnp.pad(x[:, 0].astype(jnp.int32), (0, padn))
    aidx = jnp.pad(x[:, 1].astype(jnp.int32), (0, padn))
    didx = jnp.pad(jnp.clip(node_depth[:, 0], 0, 20).astype(jnp.int32),
                   (0, padn))
    srcp = jnp.pad(edge_index[0].astype(jnp.int32),
                   (0, _EP - _E)).reshape(_EP // _ECH, _ECH)
    dstp = jnp.pad(edge_index[1].astype(jnp.int32), (0, _EP - _E),
                   constant_values=_TRASH).reshape(_EP // _ECH, _ECH)
    btp = jnp.pad(batch.astype(jnp.int32), (0, padn),
                  constant_values=_NG).reshape(_NBLK, 1, _BLK)
    zrows = jnp.zeros((_ECH, _D), f32)

    h0 = _embed_call(tidx, aidx, didx, type_emb, attr_emb, depth_emb)
    m = _mm_call(h0, ggnn_w[0])
    h = h0
    for t in range(_T):
        p0, p1 = _scatter_call(srcp, dstp, m, zrows)
        wn = ggnn_w[t + 1] if t + 1 < _T else ggnn_w[0]
        h, m = _gru_call(p0, p1, h, w_ih, w_hh,
                         b_ih.reshape(1, -1), b_hh.reshape(1, -1), wn)

    out = _final_call(h, h0, btp,
                      cl_w[:_D], cl_w[_D:], cl_b.reshape(1, -1),
                      cr_w[:_D], cr_w[_D:], cr_b.reshape(1, -1),
                      pred_w, pred_b.reshape(1, -1))
    return out


# speedup vs baseline: 1.0001x; 1.0001x over previous
import functools

import jax
import jax.numpy as jnp
from jax import lax
from jax.experimental import pallas as pl
from jax.experimental.pallas import tpu as pltpu
from jax.experimental.pallas import tpu_sc as plsc

_N = 10000
_NP = 10240
_D = 128
_T = 5
_NG = 128
_E = 320000
_EP = 327680
_NCORES = 2
_NSUB = 16
_NW = _NCORES * _NSUB
_TRASH = _NP - 8

_BLK = 1024
_NBLK = _NP // _BLK

_ECH = 128
_NCH = 64
_SR = 160


def _sc_mesh():
    return plsc.VectorSubcoreMesh(
        core_axis_name="c", subcore_axis_name="s",
        num_cores=_NCORES, num_subcores=_NSUB)


def _embed_body(tidx, aidx, didx, temb, aemb, demb, h0_out,
                idx_v, ra, rb, rc, sem):
    c = lax.axis_index("c")
    s = lax.axis_index("s")
    w = s * _NCORES + c
    per_w = _NP // _NW
    base = w * per_w

    def chunk(i, carry):
        off = base + i * _NCH
        pltpu.sync_copy(tidx.at[pl.ds(off, _NCH)], idx_v)
        pltpu.async_copy(temb.at[idx_v], ra, sem).wait()
        pltpu.sync_copy(aidx.at[pl.ds(off, _NCH)], idx_v)
        pltpu.async_copy(aemb.at[idx_v], rb, sem).wait()
        pltpu.sync_copy(didx.at[pl.ds(off, _NCH)], idx_v)
        pltpu.async_copy(demb.at[idx_v], rc, sem).wait()

        def add_b(k, carry2):
            r = k // (_D // 16)
            col = (k % (_D // 16)) * 16
            ra[r, pl.ds(col, 16)] = (ra[r, pl.ds(col, 16)]
                                     + rb[r, pl.ds(col, 16)]
                                     + rc[r, pl.ds(col, 16)])
            return carry2

        lax.fori_loop(0, _NCH * (_D // 16), add_b, 0)
        pltpu.sync_copy(ra, h0_out.at[pl.ds(off, _NCH)])
        return carry

    lax.fori_loop(0, per_w // _NCH, chunk, 0)


def _embed_call(tidx, aidx, didx, temb, aemb, demb):
    f = pl.kernel(
        _embed_body,
        out_type=jax.ShapeDtypeStruct((_NP, _D), jnp.float32),
        mesh=_sc_mesh(),
        scratch_types=[
            pltpu.VMEM((_NCH,), jnp.int32),
            pltpu.VMEM((_NCH, _D), jnp.float32),
            pltpu.VMEM((_NCH, _D), jnp.float32),
            pltpu.VMEM((_NCH, _D), jnp.float32),
            pltpu.SemaphoreType.DMA,
        ],
    )
    return f(tidx, aidx, didx, temb, aemb, demb)


def _scatter_body(src, dst, m, zrows, p0, p1,
                  acc, sidx0, sidx1, didx0, didx1, rows0, rows1, sem0, sem1):
    c = lax.axis_index("c")
    s = lax.axis_index("s")
    w = s * _NCORES + c
    zper = _NP // _NSUB
    zbase = s * zper
    nchunk = (_EP // _NW) // _ECH
    cbase = w * nchunk

    pltpu.sync_copy(zrows, rows0)

    def zb(j, carry):
        pltpu.sync_copy(rows0, acc.at[pl.ds(zbase + j * _ECH, _ECH)])
        return carry

    lax.fori_loop(0, zper // _ECH, zb, 0)
    plsc.subcore_barrier()

    pltpu.sync_copy(src.at[cbase], sidx0)
    pltpu.sync_copy(dst.at[cbase], didx0)
    pltpu.async_copy(m.at[sidx0], rows0, sem0)

    def chunk2(i, carry):
        j = i * 2
        pltpu.sync_copy(src.at[cbase + j + 1], sidx1)
        pltpu.sync_copy(dst.at[cbase + j + 1], didx1)
        pltpu.async_copy(m.at[sidx1], rows1, sem1)
        pltpu.make_async_copy(m.at[sidx0], rows0, sem0).wait()
        pltpu.sync_copy(rows0, acc.at[didx0], add=True)

        @pl.when(j + 2 < nchunk)
        def _():
            pltpu.sync_copy(src.at[cbase + j + 2], sidx0)
            pltpu.sync_copy(dst.at[cbase + j + 2], didx0)
            pltpu.async_copy(m.at[sidx0], rows0, sem0)

        pltpu.make_async_copy(m.at[sidx1], rows1, sem1).wait()
        pltpu.sync_copy(rows1, acc.at[didx1], add=True)
        return carry

    lax.fori_loop(0, nchunk // 2, chunk2, 0)
    plsc.subcore_barrier()

    def wb(j, carry):
        r0 = zbase + j * _ECH
        pltpu.sync_copy(acc.at[pl.ds(r0, _ECH)], rows0)

        @pl.when(c == 0)
        def _():
            pltpu.sync_copy(rows0, p0.at[pl.ds(r0, _ECH)])

        @pl.when(c == 1)
        def _():
            pltpu.sync_copy(rows0, p1.at[pl.ds(r0, _ECH)])

        return carry

    lax.fori_loop(0, zper // _ECH, wb, 0)


def _scatter_call(src, dst, m, zrows):
    f = pl.kernel(
        _scatter_body,
        out_type=(jax.ShapeDtypeStruct((_NP, _D), jnp.float32),
                  jax.ShapeDtypeStruct((_NP, _D), jnp.float32)),
        mesh=_sc_mesh(),
        scratch_types=[
            pltpu.VMEM_SHARED((_NP, _D), jnp.float32),
            pltpu.VMEM((_ECH,), jnp.int32),
            pltpu.VMEM((_ECH,), jnp.int32),
            pltpu.VMEM((_ECH,), jnp.int32),
            pltpu.VMEM((_ECH,), jnp.int32),
            pltpu.VMEM((_ECH, _D), jnp.float32),
            pltpu.VMEM((_ECH, _D), jnp.float32),
            pltpu.SemaphoreType.DMA,
            pltpu.SemaphoreType.DMA,
        ],
    )
    return f(src, dst, m, zrows)


def _mm_body(a, b, o):
    o[...] = jnp.dot(a[...], b[...], preferred_element_type=jnp.float32)


def _mm_call(a, b):
    return pl.pallas_call(
        _mm_body,
        grid=(_NBLK,),
        in_specs=[
            pl.BlockSpec((_BLK, _D), lambda i: (i, 0)),
            pl.BlockSpec((_D, _D), lambda i: (0, 0)),
        ],
        out_specs=pl.BlockSpec((_BLK, _D), lambda i: (i, 0)),
        out_shape=jax.ShapeDtypeStruct((_NP, _D), jnp.float32),
    )(a, b)


def _gru_body(p0, p1, h, wih, whh, bih, bhh, wn, h_out, m_out):
    agg = p0[...] + p1[...]
    gi = jnp.dot(agg, wih[...], preferred_element_type=jnp.float32) + bih[...]
    gh = jnp.dot(h[...], whh[...], preferred_element_type=jnp.float32) + bhh[...]
    r = jax.nn.sigmoid(gi[:, :_D] + gh[:, :_D])
    z = jax.nn.sigmoid(gi[:, _D:2 * _D] + gh[:, _D:2 * _D])
    n = jnp.tanh(gi[:, 2 * _D:] + r * gh[:, 2 * _D:])
    hn = (1.0 - z) * n + z * h[...]
    h_out[...] = hn
    m_out[...] = jnp.dot(hn, wn[...], preferred_element_type=jnp.float32)


def _gru_call(p0, p1, h, wih, whh, bih, bhh, wn):
    row = pl.BlockSpec((_BLK, _D), lambda i: (i, 0))
    return pl.pallas_call(
        _gru_body,
        grid=(_NBLK,),
        in_specs=[
            row, row, row,
            pl.BlockSpec((_D, 3 * _D), lambda i: (0, 0)),
            pl.BlockSpec((_D, 3 * _D), lambda i: (0, 0)),
            pl.BlockSpec((1, 3 * _D), lambda i: (0, 0)),
            pl.BlockSpec((1, 3 * _D), lambda i: (0, 0)),
            pl.BlockSpec((_D, _D), lambda i: (0, 0)),
        ],
        out_specs=(row, row),
        out_shape=(jax.ShapeDtypeStruct((_NP, _D), jnp.float32),
                   jax.ShapeDtypeStruct((_NP, _D), jnp.float32)),
    )(p0, p1, h, wih, whh, bih, bhh, wn)


def _final_body(h, h0, bt, clwh, clwh0, clb, crwh, crwh0, crb, pw, pb,
                pooled, out):
    pid = pl.program_id(0)

    @pl.when(pid == 0)
    def _():
        pooled[...] = jnp.zeros_like(pooled)

    a = (jnp.dot(h[...], clwh[...], preferred_element_type=jnp.float32)
         + jnp.dot(h0[...], clwh0[...], preferred_element_type=jnp.float32)
         + clb[...])
    b = (jnp.dot(h[...], crwh[...], preferred_element_type=jnp.float32)
         + jnp.dot(h0[...], crwh0[...], preferred_element_type=jnp.float32)
         + crb[...])
    g = jax.nn.sigmoid(a) * jnp.tanh(b)
    lbl = bt[0, 0, :]
    mask = (lbl[None, :] == lax.broadcasted_iota(
        jnp.int32, (_NG, _BLK), 0)).astype(jnp.float32)
    pooled[...] += jnp.dot(mask, g, preferred_element_type=jnp.float32)

    @pl.when(pid == pl.num_programs(0) - 1)
    def _():
        out[...] = (jnp.dot(pooled[...], pw[...],
                            preferred_element_type=jnp.float32) + pb[...])


def _final_call(h, h0, bt, clwh, clwh0, clb, crwh, crwh0, crb, pw, pb):
    row = pl.BlockSpec((_BLK, _D), lambda i: (i, 0))
    full = pl.BlockSpec((_D, _D), lambda i: (0, 0))
    bias = pl.BlockSpec((1, _D), lambda i: (0, 0))
    pooled, out = pl.pallas_call(
        _final_body,
        grid=(_NBLK,),
        in_specs=[
            row, row,
            pl.BlockSpec((1, 1, _BLK), lambda i: (i, 0, 0)),
            full, full, bias, full, full, bias, full, bias,
        ],
        out_specs=(pl.BlockSpec((_NG, _D), lambda i: (0, 0)),
                   pl.BlockSpec((_NG, _D), lambda i: (0, 0))),
        out_shape=(jax.ShapeDtypeStruct((_NG, _D), jnp.float32),
                   jax.ShapeDtypeStruct((_NG, _D), jnp.float32)),
    )(h, h0, bt, clwh, clwh0, clb, crwh, crwh0, crb, pw, pb)
    return out


def kernel(x, node_depth, edge_index, batch, type_emb, attr_emb, depth_emb,
           ggnn_w, w_ih, w_hh, b_ih, b_hh, cl_w, cl_b, cr_w, cr_b,
           pred_w, pred_b):
    f32 = jnp.float32
    padn = _NP - _N
    tidx = jnp.pad(x[:, 0].astype(jnp.int32), (0, padn))
    aidx = jnp.pad(x[:, 1].astype(jnp.int32), (0, padn))
    didx = jnp.pad(jnp.clip(node_depth[:, 0], 0, 20).astype(jnp.int32),
                   (0, padn))
    srcp = jnp.pad(edge_index[0].astype(jnp.int32),
                   (0, _EP - _E)).reshape(_EP // _ECH, _ECH)
    trash = _N + jnp.arange(_EP - _E, dtype=jnp.int32) % (_NP - _N)
    dstp = jnp.concatenate([edge_index[1].astype(jnp.int32),
                            trash]).reshape(_EP // _ECH, _ECH)
    btp = jnp.pad(batch.astype(jnp.int32), (0, padn),
                  constant_values=_NG).reshape(_NBLK, 1, _BLK)
    zrows = jnp.zeros((_ECH, _D), f32)

    h0 = _embed_call(tidx, aidx, didx, type_emb, attr_emb, depth_emb)
    m = _mm_call(h0, ggnn_w[0])
    h = h0
    for t in range(_T):
        p0, p1 = _scatter_call(srcp, dstp, m, zrows)
        wn = ggnn_w[t + 1] if t + 1 < _T else ggnn_w[0]
        h, m = _gru_call(p0, p1, h, w_ih, w_hh,
                         b_ih.reshape(1, -1), b_hh.reshape(1, -1), wn)

    out = _final_call(h, h0, btp,
                      cl_w[:_D], cl_w[_D:], cl_b.reshape(1, -1),
                      cr_w[:_D], cr_w[_D:], cr_b.reshape(1, -1),
                      pred_w, pred_b.reshape(1, -1))
    return out


# speedup vs baseline: 3.0199x; 3.0196x over previous
import functools

import jax
import jax.numpy as jnp
from jax import lax
from jax.experimental import pallas as pl
from jax.experimental.pallas import tpu as pltpu
from jax.experimental.pallas import tpu_sc as plsc

_N = 10000
_NP = 10240
_D = 128
_T = 5
_NG = 128
_E = 320000
_EP = 327680
_NCORES = 2
_NSUB = 16
_NW = _NCORES * _NSUB
_TRASH = _NP - 8

_BLK = 1024
_NBLK = _NP // _BLK

_ECH = 128
_NCH = 64
_SR = 160


def _sc_mesh():
    return plsc.VectorSubcoreMesh(
        core_axis_name="c", subcore_axis_name="s",
        num_cores=_NCORES, num_subcores=_NSUB)


def _embed_body(tidx, aidx, didx, temb, aemb, demb, h0_out,
                idx_v, ra, rb, rc, sem):
    c = lax.axis_index("c")
    s = lax.axis_index("s")
    w = s * _NCORES + c
    per_w = _NP // _NW
    base = w * per_w

    def chunk(i, carry):
        off = base + i * _NCH
        pltpu.sync_copy(tidx.at[pl.ds(off, _NCH)], idx_v)
        pltpu.async_copy(temb.at[idx_v], ra, sem).wait()
        pltpu.sync_copy(aidx.at[pl.ds(off, _NCH)], idx_v)
        pltpu.async_copy(aemb.at[idx_v], rb, sem).wait()
        pltpu.sync_copy(didx.at[pl.ds(off, _NCH)], idx_v)
        pltpu.async_copy(demb.at[idx_v], rc, sem).wait()

        def add_b(k, carry2):
            r = k // (_D // 16)
            col = (k % (_D // 16)) * 16
            ra[r, pl.ds(col, 16)] = (ra[r, pl.ds(col, 16)]
                                     + rb[r, pl.ds(col, 16)]
                                     + rc[r, pl.ds(col, 16)])
            return carry2

        lax.fori_loop(0, _NCH * (_D // 16), add_b, 0)
        pltpu.sync_copy(ra, h0_out.at[pl.ds(off, _NCH)])
        return carry

    lax.fori_loop(0, per_w // _NCH, chunk, 0)


def _embed_call(tidx, aidx, didx, temb, aemb, demb):
    f = pl.kernel(
        _embed_body,
        out_type=jax.ShapeDtypeStruct((_NP, _D), jnp.float32),
        mesh=_sc_mesh(),
        scratch_types=[
            pltpu.VMEM((_NCH,), jnp.int32),
            pltpu.VMEM((_NCH, _D), jnp.float32),
            pltpu.VMEM((_NCH, _D), jnp.float32),
            pltpu.VMEM((_NCH, _D), jnp.float32),
            pltpu.SemaphoreType.DMA,
        ],
    )
    return f(tidx, aidx, didx, temb, aemb, demb)


def _scatter_body(src, dst, m, zrows, p0, p1,
                  acc, sidx0, sidx1, didx0, didx1, rows0, rows1, sem0, sem1):
    c = lax.axis_index("c")
    s = lax.axis_index("s")
    w = s * _NCORES + c
    zper = _NP // _NSUB
    zbase = s * zper
    nchunk = (_EP // _NW) // _ECH
    cbase = w * nchunk

    pltpu.sync_copy(zrows, rows0)

    def zb(j, carry):
        pltpu.sync_copy(rows0, acc.at[pl.ds(zbase + j * _ECH, _ECH)])
        return carry

    lax.fori_loop(0, zper // _ECH, zb, 0)
    plsc.subcore_barrier()

    pltpu.sync_copy(src.at[cbase], sidx0)
    pltpu.sync_copy(dst.at[cbase], didx0)
    pltpu.async_copy(m.at[sidx0], rows0, sem0)

    def chunk2(i, carry):
        j = i * 2
        pltpu.sync_copy(src.at[cbase + j + 1], sidx1)
        pltpu.sync_copy(dst.at[cbase + j + 1], didx1)
        pltpu.async_copy(m.at[sidx1], rows1, sem1)
        pltpu.make_async_copy(m.at[sidx0], rows0, sem0).wait()
        pltpu.sync_copy(rows0, acc.at[didx0], add=True)

        @pl.when(j + 2 < nchunk)
        def _():
            pltpu.sync_copy(src.at[cbase + j + 2], sidx0)
            pltpu.sync_copy(dst.at[cbase + j + 2], didx0)
            pltpu.async_copy(m.at[sidx0], rows0, sem0)

        pltpu.make_async_copy(m.at[sidx1], rows1, sem1).wait()
        pltpu.sync_copy(rows1, acc.at[didx1], add=True)
        return carry

    lax.fori_loop(0, nchunk // 2, chunk2, 0)
    plsc.subcore_barrier()

    def wb(j, carry):
        r0 = zbase + j * _ECH
        pltpu.sync_copy(acc.at[pl.ds(r0, _ECH)], rows0)

        @pl.when(c == 0)
        def _():
            pltpu.sync_copy(rows0, p0.at[pl.ds(r0, _ECH)])

        @pl.when(c == 1)
        def _():
            pltpu.sync_copy(rows0, p1.at[pl.ds(r0, _ECH)])

        return carry

    lax.fori_loop(0, zper // _ECH, wb, 0)


def _scatter_call(src, dst, m, zrows):
    f = pl.kernel(
        _scatter_body,
        out_type=(jax.ShapeDtypeStruct((_NP, _D), jnp.float32),
                  jax.ShapeDtypeStruct((_NP, _D), jnp.float32)),
        mesh=_sc_mesh(),
        scratch_types=[
            pltpu.VMEM_SHARED((_NP, _D), jnp.float32),
            pltpu.VMEM((_ECH,), jnp.int32),
            pltpu.VMEM((_ECH,), jnp.int32),
            pltpu.VMEM((_ECH,), jnp.int32),
            pltpu.VMEM((_ECH,), jnp.int32),
            pltpu.VMEM((_ECH, _D), jnp.float32),
            pltpu.VMEM((_ECH, _D), jnp.float32),
            pltpu.SemaphoreType.DMA,
            pltpu.SemaphoreType.DMA,
        ],
    )
    return f(src, dst, m, zrows)


def _mm_body(a, b, o):
    o[...] = jnp.dot(a[...], b[...], preferred_element_type=jnp.float32)


def _mm_call(a, b):
    return pl.pallas_call(
        _mm_body,
        grid=(_NBLK,),
        in_specs=[
            pl.BlockSpec((_BLK, _D), lambda i: (i, 0)),
            pl.BlockSpec((_D, _D), lambda i: (0, 0)),
        ],
        out_specs=pl.BlockSpec((_BLK, _D), lambda i: (i, 0)),
        out_shape=jax.ShapeDtypeStruct((_NP, _D), jnp.float32),
    )(a, b)


def _gru_body(p0, p1, h, wih, whh, bih, bhh, wn, h_out, m_out):
    agg = p0[...] + p1[...]
    gi = jnp.dot(agg, wih[...], preferred_element_type=jnp.float32) + bih[...]
    gh = jnp.dot(h[...], whh[...], preferred_element_type=jnp.float32) + bhh[...]
    r = jax.nn.sigmoid(gi[:, :_D] + gh[:, :_D])
    z = jax.nn.sigmoid(gi[:, _D:2 * _D] + gh[:, _D:2 * _D])
    n = jnp.tanh(gi[:, 2 * _D:] + r * gh[:, 2 * _D:])
    hn = (1.0 - z) * n + z * h[...]
    h_out[...] = hn
    m_out[...] = jnp.dot(hn, wn[...], preferred_element_type=jnp.float32)


def _gru_call(p0, p1, h, wih, whh, bih, bhh, wn):
    row = pl.BlockSpec((_BLK, _D), lambda i: (i, 0))
    return pl.pallas_call(
        _gru_body,
        grid=(_NBLK,),
        in_specs=[
            row, row, row,
            pl.BlockSpec((_D, 3 * _D), lambda i: (0, 0)),
            pl.BlockSpec((_D, 3 * _D), lambda i: (0, 0)),
            pl.BlockSpec((1, 3 * _D), lambda i: (0, 0)),
            pl.BlockSpec((1, 3 * _D), lambda i: (0, 0)),
            pl.BlockSpec((_D, _D), lambda i: (0, 0)),
        ],
        out_specs=(row, row),
        out_shape=(jax.ShapeDtypeStruct((_NP, _D), jnp.float32),
                   jax.ShapeDtypeStruct((_NP, _D), jnp.float32)),
    )(p0, p1, h, wih, whh, bih, bhh, wn)


def _final_body(h, h0, bt, clwh, clwh0, clb, crwh, crwh0, crb, pw, pb,
                pooled, out):
    pid = pl.program_id(0)

    @pl.when(pid == 0)
    def _():
        pooled[...] = jnp.zeros_like(pooled)

    a = (jnp.dot(h[...], clwh[...], preferred_element_type=jnp.float32)
         + jnp.dot(h0[...], clwh0[...], preferred_element_type=jnp.float32)
         + clb[...])
    b = (jnp.dot(h[...], crwh[...], preferred_element_type=jnp.float32)
         + jnp.dot(h0[...], crwh0[...], preferred_element_type=jnp.float32)
         + crb[...])
    g = jax.nn.sigmoid(a) * jnp.tanh(b)
    lbl = bt[0, 0, :]
    mask = (lbl[None, :] == lax.broadcasted_iota(
        jnp.int32, (_NG, _BLK), 0)).astype(jnp.float32)
    pooled[...] += jnp.dot(mask, g, preferred_element_type=jnp.float32)

    @pl.when(pid == pl.num_programs(0) - 1)
    def _():
        out[...] = (jnp.dot(pooled[...], pw[...],
                            preferred_element_type=jnp.float32) + pb[...])


def _final_call(h, h0, bt, clwh, clwh0, clb, crwh, crwh0, crb, pw, pb):
    row = pl.BlockSpec((_BLK, _D), lambda i: (i, 0))
    full = pl.BlockSpec((_D, _D), lambda i: (0, 0))
    bias = pl.BlockSpec((1, _D), lambda i: (0, 0))
    pooled, out = pl.pallas_call(
        _final_body,
        grid=(_NBLK,),
        in_specs=[
            row, row,
            pl.BlockSpec((1, 1, _BLK), lambda i: (i, 0, 0)),
            full, full, bias, full, full, bias, full, bias,
        ],
        out_specs=(pl.BlockSpec((_NG, _D), lambda i: (0, 0)),
                   pl.BlockSpec((_NG, _D), lambda i: (0, 0))),
        out_shape=(jax.ShapeDtypeStruct((_NG, _D), jnp.float32),
                   jax.ShapeDtypeStruct((_NG, _D), jnp.float32)),
    )(h, h0, bt, clwh, clwh0, clb, crwh, crwh0, crb, pw, pb)
    return out


def kernel(x, node_depth, edge_index, batch, type_emb, attr_emb, depth_emb,
           ggnn_w, w_ih, w_hh, b_ih, b_hh, cl_w, cl_b, cr_w, cr_b,
           pred_w, pred_b):
    f32 = jnp.float32
    padn = _NP - _N
    tidx = jnp.pad(x[:, 0].astype(jnp.int32), (0, padn))
    aidx = jnp.pad(x[:, 1].astype(jnp.int32), (0, padn))
    didx = jnp.pad(jnp.clip(node_depth[:, 0], 0, 20).astype(jnp.int32),
                   (0, padn))
    padsrc = jnp.arange(_EP - _E, dtype=jnp.int32) * 37 % _NP
    srcp = jnp.concatenate([edge_index[0].astype(jnp.int32),
                            padsrc]).reshape(_EP // _ECH, _ECH)
    trash = _N + jnp.arange(_EP - _E, dtype=jnp.int32) % (_NP - _N)
    dstp = jnp.concatenate([edge_index[1].astype(jnp.int32),
                            trash]).reshape(_EP // _ECH, _ECH)
    btp = jnp.pad(batch.astype(jnp.int32), (0, padn),
                  constant_values=_NG).reshape(_NBLK, 1, _BLK)
    zrows = jnp.zeros((_ECH, _D), f32)

    h0 = _embed_call(tidx, aidx, didx, type_emb, attr_emb, depth_emb)
    m = _mm_call(h0, ggnn_w[0])
    h = h0
    for t in range(_T):
        p0, p1 = _scatter_call(srcp, dstp, m, zrows)
        wn = ggnn_w[t + 1] if t + 1 < _T else ggnn_w[0]
        h, m = _gru_call(p0, p1, h, w_ih, w_hh,
                         b_ih.reshape(1, -1), b_hh.reshape(1, -1), wn)

    out = _final_call(h, h0, btp,
                      cl_w[:_D], cl_w[_D:], cl_b.reshape(1, -1),
                      cr_w[:_D], cr_w[_D:], cr_b.reshape(1, -1),
                      pred_w, pred_b.reshape(1, -1))
    return out


# speedup vs baseline: 3.1332x; 1.0375x over previous
import functools

import jax
import jax.numpy as jnp
from jax import lax
from jax.experimental import pallas as pl
from jax.experimental.pallas import tpu as pltpu
from jax.experimental.pallas import tpu_sc as plsc

_N = 10000
_NP = 10240
_D = 128
_T = 5
_NG = 128
_E = 320000
_EP = 327680
_NCORES = 2
_NSUB = 16
_NW = _NCORES * _NSUB
_TRASH = _NP - 8

_BLK = 1024
_NBLK = _NP // _BLK

_ECH = 128
_NCH = 64
_SR = 160


def _sc_mesh():
    return plsc.VectorSubcoreMesh(
        core_axis_name="c", subcore_axis_name="s",
        num_cores=_NCORES, num_subcores=_NSUB)


def _embed_body(tidx, aidx, didx, temb, aemb, demb, h0_out,
                ti, ai, di, bufs, sems):
    c = lax.axis_index("c")
    s = lax.axis_index("s")
    w = s * _NCORES + c
    per_w = _NP // _NW
    base = w * per_w
    nch = per_w // _NCH

    def issue(i, p):
        off = base + i * _NCH
        pltpu.sync_copy(tidx.at[pl.ds(off, _NCH)], ti[p])
        pltpu.sync_copy(aidx.at[pl.ds(off, _NCH)], ai[p])
        pltpu.sync_copy(didx.at[pl.ds(off, _NCH)], di[p])
        pltpu.async_copy(temb.at[ti[p]], bufs[p][0], sems[p][0])
        pltpu.async_copy(aemb.at[ai[p]], bufs[p][1], sems[p][1])
        pltpu.async_copy(demb.at[di[p]], bufs[p][2], sems[p][2])

    def drain(i, p):
        off = base + i * _NCH
        ra, rb, rc = bufs[p]
        pltpu.make_async_copy(temb.at[ti[p]], ra, sems[p][0]).wait()
        pltpu.make_async_copy(aemb.at[ai[p]], rb, sems[p][1]).wait()
        pltpu.make_async_copy(demb.at[di[p]], rc, sems[p][2]).wait()

        def add_b(k, carry2):
            r = k // (_D // 16)
            col = (k % (_D // 16)) * 16
            ra[r, pl.ds(col, 16)] = (ra[r, pl.ds(col, 16)]
                                     + rb[r, pl.ds(col, 16)]
                                     + rc[r, pl.ds(col, 16)])
            return carry2

        lax.fori_loop(0, _NCH * (_D // 16), add_b, 0)
        pltpu.sync_copy(ra, h0_out.at[pl.ds(off, _NCH)])

    issue(0, 0)
    for i in range(nch):
        if i + 1 < nch:
            issue(i + 1, (i + 1) % 2)
        drain(i, i % 2)


def _embed_call(tidx, aidx, didx, temb, aemb, demb):
    rowbuf = pltpu.VMEM((_NCH, _D), jnp.float32)
    idxbuf = pltpu.VMEM((_NCH,), jnp.int32)
    f = pl.kernel(
        _embed_body,
        out_type=jax.ShapeDtypeStruct((_NP, _D), jnp.float32),
        mesh=_sc_mesh(),
        scratch_types=[
            [idxbuf, idxbuf],
            [idxbuf, idxbuf],
            [idxbuf, idxbuf],
            [[rowbuf, rowbuf, rowbuf], [rowbuf, rowbuf, rowbuf]],
            [[pltpu.SemaphoreType.DMA] * 3, [pltpu.SemaphoreType.DMA] * 3],
        ],
    )
    return f(tidx, aidx, didx, temb, aemb, demb)


def _scatter_body(src, dst, m, zrows, p0, p1,
                  acc, sidx0, sidx1, didx0, didx1, rows0, rows1, sem0, sem1):
    c = lax.axis_index("c")
    s = lax.axis_index("s")
    w = s * _NCORES + c
    zper = _NP // _NSUB
    zbase = s * zper
    nchunk = (_EP // _NW) // _ECH
    cbase = w * nchunk

    pltpu.sync_copy(zrows, rows0)

    def zb(j, carry):
        pltpu.sync_copy(rows0, acc.at[pl.ds(zbase + j * _ECH, _ECH)])
        return carry

    lax.fori_loop(0, zper // _ECH, zb, 0)
    plsc.subcore_barrier()

    pltpu.sync_copy(src.at[cbase], sidx0)
    pltpu.sync_copy(dst.at[cbase], didx0)
    pltpu.async_copy(m.at[sidx0], rows0, sem0)

    def chunk2(i, carry):
        j = i * 2
        pltpu.sync_copy(src.at[cbase + j + 1], sidx1)
        pltpu.sync_copy(dst.at[cbase + j + 1], didx1)
        pltpu.async_copy(m.at[sidx1], rows1, sem1)
        pltpu.make_async_copy(m.at[sidx0], rows0, sem0).wait()
        pltpu.sync_copy(rows0, acc.at[didx0], add=True)

        @pl.when(j + 2 < nchunk)
        def _():
            pltpu.sync_copy(src.at[cbase + j + 2], sidx0)
            pltpu.sync_copy(dst.at[cbase + j + 2], didx0)
            pltpu.async_copy(m.at[sidx0], rows0, sem0)

        pltpu.make_async_copy(m.at[sidx1], rows1, sem1).wait()
        pltpu.sync_copy(rows1, acc.at[didx1], add=True)
        return carry

    lax.fori_loop(0, nchunk // 2, chunk2, 0)
    plsc.subcore_barrier()

    def wb(j, carry):
        r0 = zbase + j * _ECH
        pltpu.sync_copy(acc.at[pl.ds(r0, _ECH)], rows0)

        @pl.when(c == 0)
        def _():
            pltpu.sync_copy(rows0, p0.at[pl.ds(r0, _ECH)])

        @pl.when(c == 1)
        def _():
            pltpu.sync_copy(rows0, p1.at[pl.ds(r0, _ECH)])

        return carry

    lax.fori_loop(0, zper // _ECH, wb, 0)


def _scatter_call(src, dst, m, zrows):
    f = pl.kernel(
        _scatter_body,
        out_type=(jax.ShapeDtypeStruct((_NP, _D), jnp.float32),
                  jax.ShapeDtypeStruct((_NP, _D), jnp.float32)),
        mesh=_sc_mesh(),
        scratch_types=[
            pltpu.VMEM_SHARED((_NP, _D), jnp.float32),
            pltpu.VMEM((_ECH,), jnp.int32),
            pltpu.VMEM((_ECH,), jnp.int32),
            pltpu.VMEM((_ECH,), jnp.int32),
            pltpu.VMEM((_ECH,), jnp.int32),
            pltpu.VMEM((_ECH, _D), jnp.float32),
            pltpu.VMEM((_ECH, _D), jnp.float32),
            pltpu.SemaphoreType.DMA,
            pltpu.SemaphoreType.DMA,
        ],
    )
    return f(src, dst, m, zrows)


def _mm_body(a, b, o):
    o[...] = jnp.dot(a[...], b[...], preferred_element_type=jnp.float32)


def _mm_call(a, b):
    return pl.pallas_call(
        _mm_body,
        grid=(_NBLK,),
        in_specs=[
            pl.BlockSpec((_BLK, _D), lambda i: (i, 0)),
            pl.BlockSpec((_D, _D), lambda i: (0, 0)),
        ],
        out_specs=pl.BlockSpec((_BLK, _D), lambda i: (i, 0)),
        out_shape=jax.ShapeDtypeStruct((_NP, _D), jnp.float32),
    )(a, b)


def _gru_body(p0, p1, h, wih, whh, bih, bhh, wn, h_out, m_out):
    agg = p0[...] + p1[...]
    gi = jnp.dot(agg, wih[...], preferred_element_type=jnp.float32) + bih[...]
    gh = jnp.dot(h[...], whh[...], preferred_element_type=jnp.float32) + bhh[...]
    r = jax.nn.sigmoid(gi[:, :_D] + gh[:, :_D])
    z = jax.nn.sigmoid(gi[:, _D:2 * _D] + gh[:, _D:2 * _D])
    n = jnp.tanh(gi[:, 2 * _D:] + r * gh[:, 2 * _D:])
    hn = (1.0 - z) * n + z * h[...]
    h_out[...] = hn
    m_out[...] = jnp.dot(hn, wn[...], preferred_element_type=jnp.float32)


def _gru_call(p0, p1, h, wih, whh, bih, bhh, wn):
    row = pl.BlockSpec((_BLK, _D), lambda i: (i, 0))
    return pl.pallas_call(
        _gru_body,
        grid=(_NBLK,),
        in_specs=[
            row, row, row,
            pl.BlockSpec((_D, 3 * _D), lambda i: (0, 0)),
            pl.BlockSpec((_D, 3 * _D), lambda i: (0, 0)),
            pl.BlockSpec((1, 3 * _D), lambda i: (0, 0)),
            pl.BlockSpec((1, 3 * _D), lambda i: (0, 0)),
            pl.BlockSpec((_D, _D), lambda i: (0, 0)),
        ],
        out_specs=(row, row),
        out_shape=(jax.ShapeDtypeStruct((_NP, _D), jnp.float32),
                   jax.ShapeDtypeStruct((_NP, _D), jnp.float32)),
    )(p0, p1, h, wih, whh, bih, bhh, wn)


def _final_body(h, h0, bt, clwh, clwh0, clb, crwh, crwh0, crb, pw, pb,
                pooled, out):
    pid = pl.program_id(0)

    @pl.when(pid == 0)
    def _():
        pooled[...] = jnp.zeros_like(pooled)

    a = (jnp.dot(h[...], clwh[...], preferred_element_type=jnp.float32)
         + jnp.dot(h0[...], clwh0[...], preferred_element_type=jnp.float32)
         + clb[...])
    b = (jnp.dot(h[...], crwh[...], preferred_element_type=jnp.float32)
         + jnp.dot(h0[...], crwh0[...], preferred_element_type=jnp.float32)
         + crb[...])
    g = jax.nn.sigmoid(a) * jnp.tanh(b)
    lbl = bt[0, 0, :]
    mask = (lbl[None, :] == lax.broadcasted_iota(
        jnp.int32, (_NG, _BLK), 0)).astype(jnp.float32)
    pooled[...] += jnp.dot(mask, g, preferred_element_type=jnp.float32)

    @pl.when(pid == pl.num_programs(0) - 1)
    def _():
        out[...] = (jnp.dot(pooled[...], pw[...],
                            preferred_element_type=jnp.float32) + pb[...])


def _final_call(h, h0, bt, clwh, clwh0, clb, crwh, crwh0, crb, pw, pb):
    row = pl.BlockSpec((_BLK, _D), lambda i: (i, 0))
    full = pl.BlockSpec((_D, _D), lambda i: (0, 0))
    bias = pl.BlockSpec((1, _D), lambda i: (0, 0))
    pooled, out = pl.pallas_call(
        _final_body,
        grid=(_NBLK,),
        in_specs=[
            row, row,
            pl.BlockSpec((1, 1, _BLK), lambda i: (i, 0, 0)),
            full, full, bias, full, full, bias, full, bias,
        ],
        out_specs=(pl.BlockSpec((_NG, _D), lambda i: (0, 0)),
                   pl.BlockSpec((_NG, _D), lambda i: (0, 0))),
        out_shape=(jax.ShapeDtypeStruct((_NG, _D), jnp.float32),
                   jax.ShapeDtypeStruct((_NG, _D), jnp.float32)),
    )(h, h0, bt, clwh, clwh0, clb, crwh, crwh0, crb, pw, pb)
    return out


def kernel(x, node_depth, edge_index, batch, type_emb, attr_emb, depth_emb,
           ggnn_w, w_ih, w_hh, b_ih, b_hh, cl_w, cl_b, cr_w, cr_b,
           pred_w, pred_b):
    f32 = jnp.float32
    padn = _NP - _N
    tidx = jnp.pad(x[:, 0].astype(jnp.int32), (0, padn))
    aidx = jnp.pad(x[:, 1].astype(jnp.int32), (0, padn))
    didx = jnp.pad(jnp.clip(node_depth[:, 0], 0, 20).astype(jnp.int32),
                   (0, padn))
    padsrc = jnp.arange(_EP - _E, dtype=jnp.int32) * 37 % _NP
    srcp = jnp.concatenate([edge_index[0].astype(jnp.int32),
                            padsrc]).reshape(_EP // _ECH, _ECH)
    trash = _N + jnp.arange(_EP - _E, dtype=jnp.int32) % (_NP - _N)
    dstp = jnp.concatenate([edge_index[1].astype(jnp.int32),
                            trash]).reshape(_EP // _ECH, _ECH)
    btp = jnp.pad(batch.astype(jnp.int32), (0, padn),
                  constant_values=_NG).reshape(_NBLK, 1, _BLK)
    zrows = jnp.zeros((_ECH, _D), f32)

    h0 = _embed_call(tidx, aidx, didx, type_emb, attr_emb, depth_emb)
    m = _mm_call(h0, ggnn_w[0])
    h = h0
    for t in range(_T):
        p0, p1 = _scatter_call(srcp, dstp, m, zrows)
        wn = ggnn_w[t + 1] if t + 1 < _T else ggnn_w[0]
        h, m = _gru_call(p0, p1, h, w_ih, w_hh,
                         b_ih.reshape(1, -1), b_hh.reshape(1, -1), wn)

    out = _final_call(h, h0, btp,
                      cl_w[:_D], cl_w[_D:], cl_b.reshape(1, -1),
                      cr_w[:_D], cr_w[_D:], cr_b.reshape(1, -1),
                      pred_w, pred_b.reshape(1, -1))
    return out


# speedup vs baseline: 3.1589x; 1.0082x over previous
import functools

import jax
import jax.numpy as jnp
from jax import lax
from jax.experimental import pallas as pl
from jax.experimental.pallas import tpu as pltpu
from jax.experimental.pallas import tpu_sc as plsc

_N = 10000
_NP = 10240
_D = 128
_T = 5
_NG = 128
_E = 320000
_EP = 327680
_NCORES = 2
_NSUB = 16
_NW = _NCORES * _NSUB
_TRASH = _NP - 8

_BLK = 1024
_NBLK = _NP // _BLK

_ECH = 128
_NCH = 64
_SR = 160


def _sc_mesh():
    return plsc.VectorSubcoreMesh(
        core_axis_name="c", subcore_axis_name="s",
        num_cores=_NCORES, num_subcores=_NSUB)


def _embed_body(tidx, aidx, didx, temb, aemb, demb, h0_out,
                ti, ai, di, bufs, sems):
    c = lax.axis_index("c")
    s = lax.axis_index("s")
    w = s * _NCORES + c
    per_w = _NP // _NW
    base = w * per_w
    nch = per_w // _NCH

    def issue(i, p):
        off = base + i * _NCH
        pltpu.sync_copy(tidx.at[pl.ds(off, _NCH)], ti[p])
        pltpu.sync_copy(aidx.at[pl.ds(off, _NCH)], ai[p])
        pltpu.sync_copy(didx.at[pl.ds(off, _NCH)], di[p])
        pltpu.async_copy(temb.at[ti[p]], bufs[p][0], sems[p][0])
        pltpu.async_copy(aemb.at[ai[p]], bufs[p][1], sems[p][1])
        pltpu.async_copy(demb.at[di[p]], bufs[p][2], sems[p][2])

    def drain(i, p):
        off = base + i * _NCH
        ra, rb, rc = bufs[p]
        pltpu.make_async_copy(temb.at[ti[p]], ra, sems[p][0]).wait()
        pltpu.make_async_copy(aemb.at[ai[p]], rb, sems[p][1]).wait()
        pltpu.make_async_copy(demb.at[di[p]], rc, sems[p][2]).wait()

        def add_b(k, carry2):
            r = k // (_D // 16)
            col = (k % (_D // 16)) * 16
            ra[r, pl.ds(col, 16)] = (ra[r, pl.ds(col, 16)]
                                     + rb[r, pl.ds(col, 16)]
                                     + rc[r, pl.ds(col, 16)])
            return carry2

        lax.fori_loop(0, _NCH * (_D // 16), add_b, 0)
        pltpu.sync_copy(ra, h0_out.at[pl.ds(off, _NCH)])

    issue(0, 0)
    for i in range(nch):
        if i + 1 < nch:
            issue(i + 1, (i + 1) % 2)
        drain(i, i % 2)


def _embed_call(tidx, aidx, didx, temb, aemb, demb):
    rowbuf = pltpu.VMEM((_NCH, _D), jnp.float32)
    idxbuf = pltpu.VMEM((_NCH,), jnp.int32)
    f = pl.kernel(
        _embed_body,
        out_type=jax.ShapeDtypeStruct((_NP, _D), jnp.float32),
        mesh=_sc_mesh(),
        scratch_types=[
            [idxbuf, idxbuf],
            [idxbuf, idxbuf],
            [idxbuf, idxbuf],
            [[rowbuf, rowbuf, rowbuf], [rowbuf, rowbuf, rowbuf]],
            [[pltpu.SemaphoreType.DMA] * 3, [pltpu.SemaphoreType.DMA] * 3],
        ],
    )
    return f(tidx, aidx, didx, temb, aemb, demb)


def _scatter_body(src, dst, m, zrows, p0, p1,
                  acc, sidx0, sidx1, didx0, didx1, rows0, rows1, sem0, sem1):
    c = lax.axis_index("c")
    s = lax.axis_index("s")
    w = s * _NCORES + c
    zper = _NP // _NSUB
    zbase = s * zper
    nchunk = (_EP // _NW) // _ECH
    cbase = w * nchunk

    pltpu.sync_copy(src.at[cbase], sidx0)
    pltpu.sync_copy(dst.at[cbase], didx0)
    pltpu.async_copy(m.at[sidx0], rows0, sem0)

    pltpu.sync_copy(zrows, rows1)

    def zb(j, carry):
        pltpu.sync_copy(rows1, acc.at[pl.ds(zbase + j * _ECH, _ECH)])
        return carry

    lax.fori_loop(0, zper // _ECH, zb, 0)
    plsc.subcore_barrier()


    def chunk2(i, carry):
        j = i * 2
        pltpu.sync_copy(src.at[cbase + j + 1], sidx1)
        pltpu.sync_copy(dst.at[cbase + j + 1], didx1)
        pltpu.async_copy(m.at[sidx1], rows1, sem1)
        pltpu.make_async_copy(m.at[sidx0], rows0, sem0).wait()
        pltpu.sync_copy(rows0, acc.at[didx0], add=True)

        @pl.when(j + 2 < nchunk)
        def _():
            pltpu.sync_copy(src.at[cbase + j + 2], sidx0)
            pltpu.sync_copy(dst.at[cbase + j + 2], didx0)
            pltpu.async_copy(m.at[sidx0], rows0, sem0)

        pltpu.make_async_copy(m.at[sidx1], rows1, sem1).wait()
        pltpu.sync_copy(rows1, acc.at[didx1], add=True)
        return carry

    lax.fori_loop(0, nchunk // 2, chunk2, 0)
    plsc.subcore_barrier()

    nwb = zper // _ECH
    pltpu.async_copy(acc.at[pl.ds(zbase, _ECH)], rows0, sem0)
    for j in range(nwb):
        r0 = zbase + j * _ECH
        cbuf, csem = (rows0, sem0) if j % 2 == 0 else (rows1, sem1)
        if j + 1 < nwb:
            nbuf, nsem = (rows1, sem1) if j % 2 == 0 else (rows0, sem0)
            pltpu.async_copy(acc.at[pl.ds(r0 + _ECH, _ECH)], nbuf, nsem)
        pltpu.make_async_copy(acc.at[pl.ds(r0, _ECH)], cbuf, csem).wait()

        @pl.when(c == 0)
        def _():
            pltpu.sync_copy(cbuf, p0.at[pl.ds(r0, _ECH)])

        @pl.when(c == 1)
        def _():
            pltpu.sync_copy(cbuf, p1.at[pl.ds(r0, _ECH)])


def _scatter_call(src, dst, m, zrows):
    f = pl.kernel(
        _scatter_body,
        out_type=(jax.ShapeDtypeStruct((_NP, _D), jnp.float32),
                  jax.ShapeDtypeStruct((_NP, _D), jnp.float32)),
        mesh=_sc_mesh(),
        scratch_types=[
            pltpu.VMEM_SHARED((_NP, _D), jnp.float32),
            pltpu.VMEM((_ECH,), jnp.int32),
            pltpu.VMEM((_ECH,), jnp.int32),
            pltpu.VMEM((_ECH,), jnp.int32),
            pltpu.VMEM((_ECH,), jnp.int32),
            pltpu.VMEM((_ECH, _D), jnp.float32),
            pltpu.VMEM((_ECH, _D), jnp.float32),
            pltpu.SemaphoreType.DMA,
            pltpu.SemaphoreType.DMA,
        ],
    )
    return f(src, dst, m, zrows)


def _mm_body(a, b, o):
    o[...] = jnp.dot(a[...], b[...], preferred_element_type=jnp.float32)


def _mm_call(a, b):
    return pl.pallas_call(
        _mm_body,
        grid=(_NBLK,),
        in_specs=[
            pl.BlockSpec((_BLK, _D), lambda i: (i, 0)),
            pl.BlockSpec((_D, _D), lambda i: (0, 0)),
        ],
        out_specs=pl.BlockSpec((_BLK, _D), lambda i: (i, 0)),
        out_shape=jax.ShapeDtypeStruct((_NP, _D), jnp.float32),
    )(a, b)


def _gru_body(p0, p1, h, wih, whh, bih, bhh, wn, h_out, m_out):
    agg = p0[...] + p1[...]
    gi = jnp.dot(agg, wih[...], preferred_element_type=jnp.float32) + bih[...]
    gh = jnp.dot(h[...], whh[...], preferred_element_type=jnp.float32) + bhh[...]
    r = jax.nn.sigmoid(gi[:, :_D] + gh[:, :_D])
    z = jax.nn.sigmoid(gi[:, _D:2 * _D] + gh[:, _D:2 * _D])
    n = jnp.tanh(gi[:, 2 * _D:] + r * gh[:, 2 * _D:])
    hn = (1.0 - z) * n + z * h[...]
    h_out[...] = hn
    m_out[...] = jnp.dot(hn, wn[...], preferred_element_type=jnp.float32)


def _gru_call(p0, p1, h, wih, whh, bih, bhh, wn):
    row = pl.BlockSpec((_BLK, _D), lambda i: (i, 0))
    return pl.pallas_call(
        _gru_body,
        grid=(_NBLK,),
        in_specs=[
            row, row, row,
            pl.BlockSpec((_D, 3 * _D), lambda i: (0, 0)),
            pl.BlockSpec((_D, 3 * _D), lambda i: (0, 0)),
            pl.BlockSpec((1, 3 * _D), lambda i: (0, 0)),
            pl.BlockSpec((1, 3 * _D), lambda i: (0, 0)),
            pl.BlockSpec((_D, _D), lambda i: (0, 0)),
        ],
        out_specs=(row, row),
        out_shape=(jax.ShapeDtypeStruct((_NP, _D), jnp.float32),
                   jax.ShapeDtypeStruct((_NP, _D), jnp.float32)),
    )(p0, p1, h, wih, whh, bih, bhh, wn)


def _final_body(h, h0, bt, clwh, clwh0, clb, crwh, crwh0, crb, pw, pb,
                pooled, out):
    pid = pl.program_id(0)

    @pl.when(pid == 0)
    def _():
        pooled[...] = jnp.zeros_like(pooled)

    a = (jnp.dot(h[...], clwh[...], preferred_element_type=jnp.float32)
         + jnp.dot(h0[...], clwh0[...], preferred_element_type=jnp.float32)
         + clb[...])
    b = (jnp.dot(h[...], crwh[...], preferred_element_type=jnp.float32)
         + jnp.dot(h0[...], crwh0[...], preferred_element_type=jnp.float32)
         + crb[...])
    g = jax.nn.sigmoid(a) * jnp.tanh(b)
    lbl = bt[0, 0, :]
    mask = (lbl[None, :] == lax.broadcasted_iota(
        jnp.int32, (_NG, _BLK), 0)).astype(jnp.float32)
    pooled[...] += jnp.dot(mask, g, preferred_element_type=jnp.float32)

    @pl.when(pid == pl.num_programs(0) - 1)
    def _():
        out[...] = (jnp.dot(pooled[...], pw[...],
                            preferred_element_type=jnp.float32) + pb[...])


def _final_call(h, h0, bt, clwh, clwh0, clb, crwh, crwh0, crb, pw, pb):
    row = pl.BlockSpec((_BLK, _D), lambda i: (i, 0))
    full = pl.BlockSpec((_D, _D), lambda i: (0, 0))
    bias = pl.BlockSpec((1, _D), lambda i: (0, 0))
    pooled, out = pl.pallas_call(
        _final_body,
        grid=(_NBLK,),
        in_specs=[
            row, row,
            pl.BlockSpec((1, 1, _BLK), lambda i: (i, 0, 0)),
            full, full, bias, full, full, bias, full, bias,
        ],
        out_specs=(pl.BlockSpec((_NG, _D), lambda i: (0, 0)),
                   pl.BlockSpec((_NG, _D), lambda i: (0, 0))),
        out_shape=(jax.ShapeDtypeStruct((_NG, _D), jnp.float32),
                   jax.ShapeDtypeStruct((_NG, _D), jnp.float32)),
    )(h, h0, bt, clwh, clwh0, clb, crwh, crwh0, crb, pw, pb)
    return out


def kernel(x, node_depth, edge_index, batch, type_emb, attr_emb, depth_emb,
           ggnn_w, w_ih, w_hh, b_ih, b_hh, cl_w, cl_b, cr_w, cr_b,
           pred_w, pred_b):
    f32 = jnp.float32
    padn = _NP - _N
    tidx = jnp.pad(x[:, 0].astype(jnp.int32), (0, padn))
    aidx = jnp.pad(x[:, 1].astype(jnp.int32), (0, padn))
    didx = jnp.pad(jnp.clip(node_depth[:, 0], 0, 20).astype(jnp.int32),
                   (0, padn))
    padsrc = jnp.arange(_EP - _E, dtype=jnp.int32) * 37 % _NP
    srcp = jnp.concatenate([edge_index[0].astype(jnp.int32),
                            padsrc]).reshape(_EP // _ECH, _ECH)
    trash = _N + jnp.arange(_EP - _E, dtype=jnp.int32) % (_NP - _N)
    dstp = jnp.concatenate([edge_index[1].astype(jnp.int32),
                            trash]).reshape(_EP // _ECH, _ECH)
    btp = jnp.pad(batch.astype(jnp.int32), (0, padn),
                  constant_values=_NG).reshape(_NBLK, 1, _BLK)
    zrows = jnp.zeros((_ECH, _D), f32)

    h0 = _embed_call(tidx, aidx, didx, type_emb, attr_emb, depth_emb)
    m = _mm_call(h0, ggnn_w[0])
    h = h0
    for t in range(_T):
        p0, p1 = _scatter_call(srcp, dstp, m, zrows)
        wn = ggnn_w[t + 1] if t + 1 < _T else ggnn_w[0]
        h, m = _gru_call(p0, p1, h, w_ih, w_hh,
                         b_ih.reshape(1, -1), b_hh.reshape(1, -1), wn)

    out = _final_call(h, h0, btp,
                      cl_w[:_D], cl_w[_D:], cl_b.reshape(1, -1),
                      cr_w[:_D], cr_w[_D:], cr_b.reshape(1, -1),
                      pred_w, pred_b.reshape(1, -1))
    return out
